# baseline (speedup 1.0000x reference)
import jax
import jax.numpy as jnp
from jax import lax
from jax.experimental import pallas as pl
from jax.experimental.pallas import tpu as pltpu
from jax.experimental.pallas import tpu_sc as plsc

F_TABLES = 26
B = 4096
V = 100000
D = 64
TOTAL_VALUES = 212992
N_BAGS = F_TABLES * B

NC = 2
NS = 16
NW = NC * NS

N_SPLIT = 2
F_HALF = F_TABLES // N_SPLIT
HALF_BAGS = F_HALF * B
BAGS_PER_W = HALF_BAGS // NW
N_REG = 2
BAGS_PER_REG = BAGS_PER_W // N_REG
ACC_ROWS_PER_W = BAGS_PER_REG + 8
DUMMY_OFF = BAGS_PER_REG
N_SEARCH = 10

K = 768
N_VECS = K // 16

OFF_LOAD = 1680
OFF_PAD_LEN = HALF_BAGS + (NW - 1) * BAGS_PER_W + OFF_LOAD
ZROWS = 120
NZ_DMA = ACC_ROWS_PER_W // ZROWS

OUT_CHUNK = 64
N_OUT = BAGS_PER_REG // OUT_CHUNK


def _make_body(bag_off, field_off):
  def _body(values_hbm, offsets_hbm, tables_hbm, zeros_hbm, out_hbm,
            off_v, vals_v, gidx_v, didx_v, rows_v, zeros_v, acc, sem):
    c = lax.axis_index("c")
    s = lax.axis_index("s")
    wid = c * NS + s
    bag_lo = bag_off + wid * BAGS_PER_W
    srow = s * ACC_ROWS_PER_W

    lane = lax.iota(jnp.int32, 16)

    pltpu.sync_copy(offsets_hbm.at[pl.ds(bag_lo, OFF_LOAD)], off_v)
    pltpu.sync_copy(zeros_hbm, zeros_v)

    def off_scalar(idx16):
      v = off_v[pl.ds(idx16, 16)]
      return jnp.max(jnp.where(lane == 0, v, jnp.int32(-1)))

    for h in range(N_REG):
      hbase = h * BAGS_PER_REG
      v_start = off_scalar(hbase)
      v_end = off_scalar(hbase + BAGS_PER_REG)

      for z in range(NZ_DMA):
        pltpu.sync_copy(zeros_v, acc.at[pl.ds(srow + z * ZROWS, ZROWS)])

      c0 = v_start & jnp.int32(~7)
      n_chunks = (v_end - c0 + jnp.int32(K - 1)) // jnp.int32(K)

      def chunk_body(ci, carry):
        base = pl.multiple_of(c0 + ci * jnp.int32(K), 8)
        pltpu.sync_copy(values_hbm.at[pl.ds(base, K)], vals_v)

        def vec_body(g, carry2):
          pos = base + g * 16 + lane
          vals = vals_v[pl.ds(g * 16, 16)]
          valid = (pos >= v_start) & (pos < v_end)
          lo = jnp.full((16,), hbase, jnp.int32)
          hi = jnp.full((16,), hbase + BAGS_PER_REG, jnp.int32)
          for _ in range(N_SEARCH):
            mid = (lo + hi) >> 1
            ov = plsc.load_gather(off_v, [mid])
            le = ov <= pos
            lo = jnp.where(le, mid, lo)
            hi = jnp.where(le, hi, mid)
          field = (lax.shift_right_arithmetic(bag_lo + lo, 12)
                   - jnp.int32(field_off))
          gidx = jnp.where(valid, field * jnp.int32(V) + vals,
                           wid * 47 + g * 16 + lane)
          didx = jnp.where(valid, srow + lo - hbase,
                           srow + DUMMY_OFF + (lane & 7))
          gidx_v[pl.ds(g * 16, 16)] = gidx
          didx_v[pl.ds(g * 16, 16)] = didx
          return carry2

        lax.fori_loop(0, N_VECS, vec_body, 0)
        pltpu.async_copy(tables_hbm.at[gidx_v], rows_v, sem).wait()
        pltpu.sync_copy(rows_v, acc.at[didx_v], add=True)
        return carry

      lax.fori_loop(0, n_chunks, chunk_body, 0)

      for k in range(N_OUT):
        gb = bag_lo + hbase + k * OUT_CHUNK
        f = (lax.shift_right_arithmetic(gb, 12) - jnp.int32(field_off))
        b0 = lax.bitwise_and(gb, jnp.int32(B - 1))
        pltpu.sync_copy(acc.at[pl.ds(srow + k * OUT_CHUNK, OUT_CHUNK)],
                        out_hbm.at[pl.ds(b0, OUT_CHUNK),
                                   pl.ds(pl.multiple_of(f * D, D), D)])

  return _body


def _make_run(bag_off, field_off):
  mesh = plsc.VectorSubcoreMesh(core_axis_name="c", subcore_axis_name="s")
  return pl.kernel(
      _make_body(bag_off, field_off),
      out_type=jax.ShapeDtypeStruct((B, F_HALF * D), jnp.float32),
      mesh=mesh,
      compiler_params=pltpu.CompilerParams(
          needs_layout_passes=False, use_tc_tiling_on_sc=False),
      scratch_types=[
          pltpu.VMEM((OFF_LOAD,), jnp.int32),
          pltpu.VMEM((K,), jnp.int32),
          pltpu.VMEM((K,), jnp.int32),
          pltpu.VMEM((K,), jnp.int32),
          pltpu.VMEM((K, D), jnp.float32),
          pltpu.VMEM((ZROWS, D), jnp.float32),
          pltpu.VMEM_SHARED((NS * ACC_ROWS_PER_W, D), jnp.float32),
          pltpu.SemaphoreType.DMA,
      ],
  )


@jax.jit
def kernel(values, offsets, tables):
  values_pad = jnp.concatenate([values, jnp.zeros((K,), jnp.int32)])
  offsets_pad = jnp.concatenate(
      [offsets,
       jnp.full((OFF_PAD_LEN - (N_BAGS + 1),), TOTAL_VALUES, jnp.int32)])
  zeros = jnp.zeros((ZROWS, D), jnp.float32)

  outs = []
  for sp in range(N_SPLIT):
    t = tables[sp * F_HALF:(sp + 1) * F_HALF].reshape(F_HALF * V, D)
    run = _make_run(sp * HALF_BAGS, sp * F_HALF)
    outs.append(run(values_pad, offsets_pad, t, zeros))
  return jnp.concatenate(outs, axis=1)


# speedup vs baseline: 1.4742x; 1.4742x over previous
import jax
import jax.numpy as jnp
from jax import lax
from jax.experimental import pallas as pl
from jax.experimental.pallas import tpu as pltpu
from jax.experimental.pallas import tpu_sc as plsc

F_TABLES = 26
B = 4096
V = 100000
D = 64
TOTAL_VALUES = 212992
N_BAGS = F_TABLES * B

NC = 2
NS = 16
NW = NC * NS

BAGS_PER_W = N_BAGS // NW
N_REG = 4
BAGS_PER_REG = BAGS_PER_W // N_REG
ACC_ROWS_PER_W = BAGS_PER_REG + 8
DUMMY_OFF = BAGS_PER_REG
N_SEARCH = 10

K = 768
ROWS_PER_DMA = 128
N_DMA = K // ROWS_PER_DMA
VECS_PER_DMA = ROWS_PER_DMA // 16
N_VECS = K // 16

OFF_LOAD = 3344
OFF_PAD_LEN = (NW - 1) * BAGS_PER_W + OFF_LOAD
ZROWS = 120
NZ_DMA = ACC_ROWS_PER_W // ZROWS

OUT_CHUNK = 64
N_OUT = BAGS_PER_REG // OUT_CHUNK


def _body(values_hbm, offsets_hbm, tables_hbm, zeros_hbm, out_hbm,
          off_v, vals_v, gidx_v, didx_v, rows_v, zeros_v, acc, sem):
  c = lax.axis_index("c")
  s = lax.axis_index("s")
  wid = c * NS + s
  bag_lo = wid * BAGS_PER_W
  srow = s * ACC_ROWS_PER_W

  lane = lax.iota(jnp.int32, 16)

  pltpu.sync_copy(offsets_hbm.at[pl.ds(bag_lo, OFF_LOAD)], off_v)
  pltpu.sync_copy(zeros_hbm, zeros_v)

  def off_scalar(idx16):
    v = off_v[pl.ds(idx16, 16)]
    return jnp.max(jnp.where(lane == 0, v, jnp.int32(-1)))

  for h in range(N_REG):
    hbase = h * BAGS_PER_REG
    v_start = off_scalar(hbase)
    v_end = off_scalar(hbase + BAGS_PER_REG)

    for z in range(NZ_DMA):
      pltpu.sync_copy(zeros_v, acc.at[pl.ds(srow + z * ZROWS, ZROWS)])

    c0 = v_start & jnp.int32(~7)
    n_chunks = (v_end - c0 + jnp.int32(K - 1)) // jnp.int32(K)

    def chunk_body(ci, carry):
      base = pl.multiple_of(c0 + ci * jnp.int32(K), 8)
      pltpu.sync_copy(values_hbm.at[pl.ds(base, K)], vals_v)

      def vec_body(g, carry2):
          pos = base + g * 16 + lane
          vals = vals_v[pl.ds(g * 16, 16)]
          valid = (pos >= v_start) & (pos < v_end)
          lo = jnp.full((16,), hbase, jnp.int32)
          hi = jnp.full((16,), hbase + BAGS_PER_REG, jnp.int32)
          for _ in range(N_SEARCH):
            mid = (lo + hi) >> 1
            ov = plsc.load_gather(off_v, [mid])
            le = ov <= pos
            lo = jnp.where(le, mid, lo)
            hi = jnp.where(le, hi, mid)
          field = lax.shift_right_arithmetic(bag_lo + lo, 12)
          gidx = jnp.where(valid, field * jnp.int32(V) + vals,
                           wid * 47 + g * 16 + lane)
          didx = jnp.where(valid, srow + lo - hbase,
                           srow + DUMMY_OFF + (lane & 7))
          gidx_v[pl.ds(g * 16, 16)] = gidx
          didx_v[pl.ds(g * 16, 16)] = didx
          return carry2
      lax.fori_loop(0, N_VECS, vec_body, 0)
      pltpu.async_copy(tables_hbm.at[gidx_v], rows_v, sem).wait()
      pltpu.sync_copy(rows_v, acc.at[didx_v], add=True)
      return carry

    lax.fori_loop(0, n_chunks, chunk_body, 0)

    for k in range(N_OUT):
      gb = bag_lo + hbase + k * OUT_CHUNK
      f = lax.shift_right_arithmetic(gb, 12)
      b0 = lax.bitwise_and(gb, jnp.int32(B - 1))
      pltpu.sync_copy(acc.at[pl.ds(srow + k * OUT_CHUNK, OUT_CHUNK)],
                      out_hbm.at[pl.ds(b0, OUT_CHUNK),
                                 pl.ds(pl.multiple_of(f * D, D), D)])


@jax.jit
def kernel(values, offsets, tables):
  values_pad = jnp.concatenate([values, jnp.zeros((K,), jnp.int32)])
  offsets_pad = jnp.concatenate(
      [offsets,
       jnp.full((OFF_PAD_LEN - (N_BAGS + 1),), TOTAL_VALUES, jnp.int32)])
  tables_flat = tables.reshape(F_TABLES * V, D)
  zeros = jnp.zeros((ZROWS, D), jnp.float32)

  mesh = plsc.VectorSubcoreMesh(core_axis_name="c", subcore_axis_name="s")
  run = pl.kernel(
      _body,
      out_type=jax.ShapeDtypeStruct((B, F_TABLES * D), jnp.float32),
      mesh=mesh,
      compiler_params=pltpu.CompilerParams(
          needs_layout_passes=False, use_tc_tiling_on_sc=False),
      scratch_types=[
          pltpu.VMEM((OFF_LOAD,), jnp.int32),
          pltpu.VMEM((K,), jnp.int32),
          pltpu.VMEM((K,), jnp.int32),
          pltpu.VMEM((K,), jnp.int32),
          pltpu.VMEM((K, D), jnp.float32),
          pltpu.VMEM((ZROWS, D), jnp.float32),
          pltpu.VMEM_SHARED((NS * ACC_ROWS_PER_W, D), jnp.float32),
          pltpu.SemaphoreType.DMA,
      ],
  )
  return run(values_pad, offsets_pad, tables_flat, zeros)


# speedup vs baseline: 1.4859x; 1.0079x over previous
import jax
import jax.numpy as jnp
from jax import lax
from jax.experimental import pallas as pl
from jax.experimental.pallas import tpu as pltpu
from jax.experimental.pallas import tpu_sc as plsc

F_TABLES = 26
B = 4096
V = 100000
D = 64
TOTAL_VALUES = 212992
N_BAGS = F_TABLES * B

NC = 2
NS = 16
NW = NC * NS

BAGS_PER_W = N_BAGS // NW
N_REG = 4
BAGS_PER_REG = BAGS_PER_W // N_REG
ACC_ROWS_PER_W = BAGS_PER_REG + 8
DUMMY_OFF = BAGS_PER_REG
N_SEARCH = 10

K = 768
ROWS_PER_DMA = 128
N_DMA = K // ROWS_PER_DMA
VECS_PER_DMA = ROWS_PER_DMA // 16
N_VECS = K // 16

OFF_LOAD = 3344
OFF_PAD_LEN = (NW - 1) * BAGS_PER_W + OFF_LOAD
ZROWS = 120
NZ_DMA = ACC_ROWS_PER_W // ZROWS

OUT_CHUNK = 64
N_OUT = BAGS_PER_REG // OUT_CHUNK


def _body(values_hbm, offsets_hbm, tables_hbm, zeros_hbm, out_hbm,
          off_v, vals_v, gidx_v, didxa_v, didxb_v, rows_v, zeros_v, acc,
          sem, sema, semb):
  c = lax.axis_index("c")
  s = lax.axis_index("s")
  wid = c * NS + s
  bag_lo = wid * BAGS_PER_W
  srow = s * ACC_ROWS_PER_W

  lane = lax.iota(jnp.int32, 16)

  pltpu.sync_copy(offsets_hbm.at[pl.ds(bag_lo, OFF_LOAD)], off_v)
  pltpu.sync_copy(zeros_hbm, zeros_v)

  def off_scalar(idx16):
    v = off_v[pl.ds(idx16, 16)]
    return jnp.max(jnp.where(lane == 0, v, jnp.int32(-1)))

  for h in range(N_REG):
    hbase = h * BAGS_PER_REG
    v_start = off_scalar(hbase)
    v_end = off_scalar(hbase + BAGS_PER_REG)

    for z in range(NZ_DMA):
      pltpu.sync_copy(zeros_v, acc.at[pl.ds(srow + z * ZROWS, ZROWS)])

    c0 = v_start & jnp.int32(~7)
    n_chunks = (v_end - c0 + jnp.int32(K - 1)) // jnp.int32(K)

    def make_half(didx_ref, my_sem, other_sem):
      def half(ci, first):
        base = pl.multiple_of(c0 + ci * jnp.int32(K), 8)
        pltpu.sync_copy(values_hbm.at[pl.ds(base, K)], vals_v)

        def vec_body(g, carry2):
          pos = base + g * 16 + lane
          vals = vals_v[pl.ds(g * 16, 16)]
          valid = (pos >= v_start) & (pos < v_end)
          lo = jnp.full((16,), hbase, jnp.int32)
          hi = jnp.full((16,), hbase + BAGS_PER_REG, jnp.int32)
          for _ in range(N_SEARCH):
            mid = (lo + hi) >> 1
            ov = plsc.load_gather(off_v, [mid])
            le = ov <= pos
            lo = jnp.where(le, mid, lo)
            hi = jnp.where(le, hi, mid)
          field = lax.shift_right_arithmetic(bag_lo + lo, 12)
          gidx = jnp.where(valid, field * jnp.int32(V) + vals,
                           wid * 47 + g * 16 + lane)
          didx = jnp.where(valid, srow + lo - hbase,
                           srow + DUMMY_OFF + (lane & 7))
          gidx_v[pl.ds(g * 16, 16)] = gidx
          didx_ref[pl.ds(g * 16, 16)] = didx
          return carry2

        lax.fori_loop(0, N_VECS, vec_body, 0)

        @pl.when(jnp.logical_not(first))
        def _():
          pltpu.make_async_copy(rows_v, acc.at[didx_ref], other_sem).wait()

        pltpu.async_copy(tables_hbm.at[gidx_v], rows_v, sem).wait()
        pltpu.async_copy(rows_v, acc.at[didx_ref], my_sem)
      return half

    half_a = make_half(didxa_v, sema, semb)
    half_b = make_half(didxb_v, semb, sema)

    def pair_body(ci2, carry):
      half_a(2 * ci2, ci2 == 0)

      @pl.when(2 * ci2 + 1 < n_chunks)
      def _():
        half_b(2 * ci2 + 1, jnp.bool_(False))
      return carry

    lax.fori_loop(0, (n_chunks + jnp.int32(1)) // jnp.int32(2), pair_body, 0)

    odd = lax.bitwise_and(n_chunks, 1) == 1

    @pl.when(odd)
    def _():
      pltpu.make_async_copy(rows_v, acc.at[didxa_v], sema).wait()

    @pl.when((n_chunks > 0) & jnp.logical_not(odd))
    def _():
      pltpu.make_async_copy(rows_v, acc.at[didxb_v], semb).wait()

    for k in range(N_OUT):
      gb = bag_lo + hbase + k * OUT_CHUNK
      f = lax.shift_right_arithmetic(gb, 12)
      b0 = lax.bitwise_and(gb, jnp.int32(B - 1))
      pltpu.sync_copy(acc.at[pl.ds(srow + k * OUT_CHUNK, OUT_CHUNK)],
                      out_hbm.at[pl.ds(b0, OUT_CHUNK),
                                 pl.ds(pl.multiple_of(f * D, D), D)])


@jax.jit
def kernel(values, offsets, tables):
  values_pad = jnp.concatenate([values, jnp.zeros((K,), jnp.int32)])
  offsets_pad = jnp.concatenate(
      [offsets,
       jnp.full((OFF_PAD_LEN - (N_BAGS + 1),), TOTAL_VALUES, jnp.int32)])
  tables_flat = tables.reshape(F_TABLES * V, D)
  zeros = jnp.zeros((ZROWS, D), jnp.float32)

  mesh = plsc.VectorSubcoreMesh(core_axis_name="c", subcore_axis_name="s")
  run = pl.kernel(
      _body,
      out_type=jax.ShapeDtypeStruct((B, F_TABLES * D), jnp.float32),
      mesh=mesh,
      compiler_params=pltpu.CompilerParams(
          needs_layout_passes=False, use_tc_tiling_on_sc=False),
      scratch_types=[
          pltpu.VMEM((OFF_LOAD,), jnp.int32),
          pltpu.VMEM((K,), jnp.int32),
          pltpu.VMEM((K,), jnp.int32),
          pltpu.VMEM((K,), jnp.int32),
          pltpu.VMEM((K,), jnp.int32),
          pltpu.VMEM((K, D), jnp.float32),
          pltpu.VMEM((ZROWS, D), jnp.float32),
          pltpu.VMEM_SHARED((NS * ACC_ROWS_PER_W, D), jnp.float32),
          pltpu.SemaphoreType.DMA,
          pltpu.SemaphoreType.DMA,
          pltpu.SemaphoreType.DMA,
      ],
  )
  return run(values_pad, offsets_pad, tables_flat, zeros)
